# Initial kernel scaffold; baseline (speedup 1.0000x reference)
import jax
import jax.numpy as jnp
from jax.experimental import pallas as pl


def kernel(data, codebook, indices):
    raise NotImplementedError("write your pallas kernel here")


# speedup vs baseline: 2.4874x; 2.4874x over previous
import functools

import jax
import jax.numpy as jnp
from jax import lax
from jax.experimental import pallas as pl
from jax.experimental.pallas import tpu as pltpu
from jax.experimental.pallas import tpu_sc as plsc

D = 16
NC, NS = 2, 16
NW = NC * NS
C = 2048
G = 128


def _make_gather(B):
    assert B % (NW * C) == 0
    b_per_w = B // NW
    n_chunks = b_per_w // C
    mesh = plsc.VectorSubcoreMesh(core_axis_name="c", subcore_axis_name="s")

    @functools.partial(
        pl.kernel,
        mesh=mesh,
        out_type=jax.ShapeDtypeStruct((B, D), jnp.float32),
        scratch_types=[
            pltpu.VMEM((C,), jnp.int32),
            pltpu.VMEM((C, D), jnp.float32),
            pltpu.SemaphoreType.DMA,
        ],
        compiler_params=pltpu.CompilerParams(use_tc_tiling_on_sc=False),
    )
    def k(cb_hbm, idx_hbm, out_hbm, idx_v, rows_v, sem):
        wid = lax.axis_index("s") * NC + lax.axis_index("c")
        base = wid * b_per_w

        def body(step, carry):
            off = base + step * C
            pltpu.sync_copy(idx_hbm.at[pl.ds(off, C)], idx_v)
            copies = []
            for j in range(C // G):
                copies.append(pltpu.async_copy(
                    cb_hbm.at[idx_v.at[pl.ds(j * G, G)]],
                    rows_v.at[pl.ds(j * G, G)],
                    sem,
                ))
            for cp in copies:
                cp.wait()
            pltpu.sync_copy(rows_v, out_hbm.at[pl.ds(off, C)])
            return carry

        lax.fori_loop(0, n_chunks, body, 0)

    return k


def kernel(data, codebook, indices):
    del data
    B = indices.size
    idx_flat = indices.reshape(B).astype(jnp.int32)
    out = _make_gather(B)(codebook, idx_flat)
    return out.reshape(*indices.shape, codebook.shape[1])


# speedup vs baseline: 2.4892x; 1.0007x over previous
import functools

import jax
import jax.numpy as jnp
from jax import lax
from jax.experimental import pallas as pl
from jax.experimental.pallas import tpu as pltpu
from jax.experimental.pallas import tpu_sc as plsc

D = 16
NC, NS = 2, 16
NW = NC * NS
C = 2048
G = 2048


def _make_gather(B):
    assert B % (NW * C) == 0
    b_per_w = B // NW
    n_chunks = b_per_w // C
    mesh = plsc.VectorSubcoreMesh(core_axis_name="c", subcore_axis_name="s")

    @functools.partial(
        pl.kernel,
        mesh=mesh,
        out_type=jax.ShapeDtypeStruct((B, D), jnp.float32),
        scratch_types=[
            pltpu.VMEM((C,), jnp.int32),
            pltpu.VMEM((C, D), jnp.float32),
            pltpu.SemaphoreType.DMA,
        ],
        compiler_params=pltpu.CompilerParams(use_tc_tiling_on_sc=False),
    )
    def k(cb_hbm, idx_hbm, out_hbm, idx_v, rows_v, sem):
        wid = lax.axis_index("s") * NC + lax.axis_index("c")
        base = wid * b_per_w

        def body(step, carry):
            off = base + step * C
            pltpu.sync_copy(idx_hbm.at[pl.ds(off, C)], idx_v)
            copies = []
            for j in range(C // G):
                copies.append(pltpu.async_copy(
                    cb_hbm.at[idx_v.at[pl.ds(j * G, G)]],
                    rows_v.at[pl.ds(j * G, G)],
                    sem,
                ))
            for cp in copies:
                cp.wait()
            pltpu.sync_copy(rows_v, out_hbm.at[pl.ds(off, C)])
            return carry

        lax.fori_loop(0, n_chunks, body, 0)

    return k


def kernel(data, codebook, indices):
    del data
    B = indices.size
    idx_flat = indices.reshape(B).astype(jnp.int32)
    out = _make_gather(B)(codebook, idx_flat)
    return out.reshape(*indices.shape, codebook.shape[1])


# speedup vs baseline: 4.6656x; 1.8743x over previous
import functools

import jax
import jax.numpy as jnp
from jax import lax
from jax.experimental import pallas as pl
from jax.experimental.pallas import tpu as pltpu
from jax.experimental.pallas import tpu_sc as plsc

D = 16
NC, NS = 2, 16
NW = NC * NS
TPW = 128 // NW
NJ = 200
STEPS = NJ * TPW


def _make_gather():
    mesh = plsc.VectorSubcoreMesh(core_axis_name="c", subcore_axis_name="s")

    @functools.partial(
        pl.kernel,
        mesh=mesh,
        out_type=jax.ShapeDtypeStruct((NJ, 2, 128, 8, 128), jnp.float32),
        scratch_types=[
            pltpu.VMEM((NJ // 8, TPW, 8, 128), jnp.int32),
            pltpu.VMEM((128, D), jnp.float32),
            pltpu.VMEM((128, D), jnp.float32),
            pltpu.VMEM((2, 8, 128), jnp.float32),
            pltpu.VMEM((2, 8, 128), jnp.float32),
            pltpu.SemaphoreType.DMA,
            pltpu.SemaphoreType.DMA,
            pltpu.SemaphoreType.DMA,
            pltpu.SemaphoreType.DMA,
        ],
        compiler_params=pltpu.CompilerParams(
            use_tc_tiling_on_sc=False, needs_layout_passes=False),
    )
    def k(cb_hbm, idx_hbm, out_hbm, idx_v, rows0, rows1, tr0, tr1,
          sg0, sg1, so0, so1):
        rows = (rows0, rows1)
        trs = (tr0, tr1)
        sgs = (sg0, sg1)
        sos = (so0, so1)
        wid = lax.axis_index("s") * NC + lax.axis_index("c")
        t0 = wid * TPW

        def idx_slice(step):
            jl = step % NJ
            tl = step // NJ
            return idx_v.at[jl // 8, tl, jl % 8]

        pltpu.sync_copy(idx_hbm.at[:, pl.ds(t0, TPW)], idx_v)
        pltpu.async_copy(cb_hbm.at[idx_slice(0)], rows0, sg0)
        pltpu.async_copy(cb_hbm.at[idx_slice(1)], rows1, sg1)

        lanes = lax.iota(jnp.int32, 16)

        def body(outer, carry):
            for b in range(2):
                step = outer * 2 + b
                jl = step % NJ
                tg = t0 + step // NJ
                pltpu.make_async_copy(
                    cb_hbm.at[pl.ds(0, 128)], rows[b], sgs[b]).wait()
                @pl.when(step >= 2)
                def _():
                    pltpu.make_async_copy(
                        trs[b], out_hbm.at[0, :, 0], sos[b]).wait()
                for kg in range(2):
                    for ks in range(8):
                        colv = jnp.full((16,), kg * 8 + ks, jnp.int32)
                        for il0 in range(8):
                            v = plsc.load_gather(
                                rows[b], [lanes + il0 * 16, colv])
                            trs[b][kg, ks, pl.ds(il0 * 16, 16)] = v
                @pl.when(step + 2 < STEPS)
                def _():
                    pltpu.async_copy(
                        cb_hbm.at[idx_slice(step + 2)], rows[b], sgs[b])
                pltpu.async_copy(trs[b], out_hbm.at[jl, :, tg], sos[b])
            return carry

        lax.fori_loop(0, STEPS // 2, body, 0)
        pltpu.make_async_copy(tr0, out_hbm.at[0, :, 0], so0).wait()
        pltpu.make_async_copy(tr1, out_hbm.at[0, :, 0], so1).wait()

    return k


def kernel(data, codebook, indices):
    del data
    idx4 = (indices.astype(jnp.int32).T
            .reshape(NJ // 8, 8, 128, 128).transpose(0, 2, 1, 3))
    a = _make_gather()(codebook, idx4)
    return a.transpose(2, 4, 0, 1, 3).reshape(16384, NJ, D)


# speedup vs baseline: 5.9173x; 1.2683x over previous
import functools

import jax
import jax.numpy as jnp
from jax import lax
from jax.experimental import pallas as pl
from jax.experimental.pallas import tpu as pltpu
from jax.experimental.pallas import tpu_sc as plsc

D = 16
NC, NS = 2, 16
NW = NC * NS
TPW = 128 // NW
NJ = 200
STEPS = NJ * TPW


def _make_gather():
    mesh = plsc.VectorSubcoreMesh(core_axis_name="c", subcore_axis_name="s")

    @functools.partial(
        pl.kernel,
        mesh=mesh,
        out_type=jax.ShapeDtypeStruct((NJ, 2, 128, 8, 128), jnp.float32),
        scratch_types=[
            pltpu.VMEM((NJ // 8, TPW, 8, 128), jnp.int32),
            pltpu.VMEM((128, D), jnp.float32),
            pltpu.VMEM((128, D), jnp.float32),
            pltpu.VMEM((2, 8, 128), jnp.float32),
            pltpu.VMEM((2, 8, 128), jnp.float32),
            pltpu.SemaphoreType.DMA,
            pltpu.SemaphoreType.DMA,
            pltpu.SemaphoreType.DMA,
            pltpu.SemaphoreType.DMA,
        ],
        compiler_params=pltpu.CompilerParams(
            use_tc_tiling_on_sc=False, needs_layout_passes=False),
    )
    def k(cb_hbm, idx_hbm, out_hbm, idx_v, rows0, rows1, tr0, tr1,
          sg0, sg1, so0, so1):
        rows = (rows0, rows1)
        trs = (tr0, tr1)
        sgs = (sg0, sg1)
        sos = (so0, so1)
        wid = lax.axis_index("s") * NC + lax.axis_index("c")
        t0 = wid * TPW

        def idx_slice(step):
            jl = step % NJ
            tl = step // NJ
            return idx_v.at[jl // 8, tl, jl % 8]

        pltpu.sync_copy(idx_hbm.at[:, pl.ds(t0, TPW)], idx_v)
        pltpu.async_copy(cb_hbm.at[idx_slice(0)], rows0, sg0)
        pltpu.async_copy(cb_hbm.at[idx_slice(1)], rows1, sg1)

        lanes = lax.iota(jnp.int32, 16)

        def body(outer, carry):
            for b in range(2):
                step = outer * 2 + b
                jl = step % NJ
                tg = t0 + step // NJ
                pltpu.make_async_copy(
                    cb_hbm.at[pl.ds(0, 128)], rows[b], sgs[b]).wait()
                @pl.when(step >= 2)
                def _():
                    pltpu.make_async_copy(
                        trs[b], out_hbm.at[0, :, 0], sos[b]).wait()
                rb, tb = rows[b], trs[b]

                @plsc.parallel_loop(0, 128, 1, unroll=8)
                def _(i):
                    kk = i // 8
                    il0 = i % 8
                    v = plsc.load_gather(
                        rb, [lanes + il0 * 16, jnp.full((16,), kk, jnp.int32)])
                    tb[kk // 8, kk % 8, pl.ds(il0 * 16, 16)] = v
                @pl.when(step + 2 < STEPS)
                def _():
                    pltpu.async_copy(
                        cb_hbm.at[idx_slice(step + 2)], rows[b], sgs[b])
                pltpu.async_copy(trs[b], out_hbm.at[jl, :, tg], sos[b])
            return carry

        lax.fori_loop(0, STEPS // 2, body, 0)
        pltpu.make_async_copy(tr0, out_hbm.at[0, :, 0], so0).wait()
        pltpu.make_async_copy(tr1, out_hbm.at[0, :, 0], so1).wait()

    return k


def kernel(data, codebook, indices):
    del data
    idx4 = (indices.astype(jnp.int32).T
            .reshape(NJ // 8, 8, 128, 128).transpose(0, 2, 1, 3))
    a = _make_gather()(codebook, idx4)
    return a.transpose(2, 4, 0, 1, 3).reshape(16384, NJ, D)


# speedup vs baseline: 5.9973x; 1.0135x over previous
import functools

import jax
import jax.numpy as jnp
from jax import lax
from jax.experimental import pallas as pl
from jax.experimental.pallas import tpu as pltpu
from jax.experimental.pallas import tpu_sc as plsc

D = 16
NC, NS = 2, 16
NW = NC * NS
TPW = 128 // NW
NJ = 200
STEPS = NJ * TPW


def _make_gather():
    mesh = plsc.VectorSubcoreMesh(core_axis_name="c", subcore_axis_name="s")

    @functools.partial(
        pl.kernel,
        mesh=mesh,
        out_type=jax.ShapeDtypeStruct((NJ, 2, 128, 8, 128), jnp.float32),
        scratch_types=[
            pltpu.VMEM((NJ // 8, TPW, 8, 128), jnp.int32),
            pltpu.VMEM((128, D), jnp.float32),
            pltpu.VMEM((128, D), jnp.float32),
            pltpu.VMEM((2, 8, 128), jnp.float32),
            pltpu.VMEM((2, 8, 128), jnp.float32),
            pltpu.SemaphoreType.DMA,
            pltpu.SemaphoreType.DMA,
            pltpu.SemaphoreType.DMA,
            pltpu.SemaphoreType.DMA,
        ],
        compiler_params=pltpu.CompilerParams(
            use_tc_tiling_on_sc=False, needs_layout_passes=False),
    )
    def k(cb_hbm, idx_hbm, out_hbm, idx_v, rows0, rows1, tr0, tr1,
          sg0, sg1, so0, so1):
        rows = (rows0, rows1)
        trs = (tr0, tr1)
        sgs = (sg0, sg1)
        sos = (so0, so1)
        wid = lax.axis_index("s") * NC + lax.axis_index("c")
        t0 = wid * TPW

        def idx_slice(step):
            jl = step % NJ
            tl = step // NJ
            return idx_v.at[jl // 8, tl, jl % 8]

        pltpu.sync_copy(idx_hbm.at[:, pl.ds(t0, TPW)], idx_v)
        pltpu.async_copy(cb_hbm.at[idx_slice(0)], rows0, sg0)
        pltpu.async_copy(cb_hbm.at[idx_slice(1)], rows1, sg1)

        lanes = lax.iota(jnp.int32, 16)

        def body(outer, carry):
            for b in range(2):
                step = outer * 2 + b
                jl = step % NJ
                tg = t0 + step // NJ
                pltpu.make_async_copy(
                    cb_hbm.at[pl.ds(0, 128)], rows[b], sgs[b]).wait()
                @pl.when(step >= 2)
                def _():
                    pltpu.make_async_copy(
                        trs[b], out_hbm.at[0, :, 0], sos[b]).wait()
                rb, tb = rows[b], trs[b]

                @plsc.parallel_loop(0, 128, 1, unroll=16)
                def _(i):
                    kk = i // 8
                    il0 = i % 8
                    v = plsc.load_gather(
                        rb, [lanes + il0 * 16, jnp.full((16,), kk, jnp.int32)])
                    tb[kk // 8, kk % 8, pl.ds(il0 * 16, 16)] = v
                @pl.when(step + 2 < STEPS)
                def _():
                    pltpu.async_copy(
                        cb_hbm.at[idx_slice(step + 2)], rows[b], sgs[b])
                pltpu.async_copy(trs[b], out_hbm.at[jl, :, tg], sos[b])
            return carry

        lax.fori_loop(0, STEPS // 2, body, 0)
        pltpu.make_async_copy(tr0, out_hbm.at[0, :, 0], so0).wait()
        pltpu.make_async_copy(tr1, out_hbm.at[0, :, 0], so1).wait()

    return k


def kernel(data, codebook, indices):
    del data
    idx4 = (indices.astype(jnp.int32).T
            .reshape(NJ // 8, 8, 128, 128).transpose(0, 2, 1, 3))
    a = _make_gather()(codebook, idx4)
    return a.transpose(2, 4, 0, 1, 3).reshape(16384, NJ, D)


# speedup vs baseline: 7.0300x; 1.1722x over previous
import functools

import jax
import jax.numpy as jnp
from jax import lax
from jax.experimental import pallas as pl
from jax.experimental.pallas import tpu as pltpu
from jax.experimental.pallas import tpu_sc as plsc

D = 16
NC, NS = 2, 16
NW = NC * NS
TPW = 128 // NW
NJ = 200
STEPS = NJ * TPW


def _make_gather():
    mesh = plsc.VectorSubcoreMesh(core_axis_name="c", subcore_axis_name="s")

    @functools.partial(
        pl.kernel,
        mesh=mesh,
        out_type=jax.ShapeDtypeStruct((NJ, 2, 128, 8, 128), jnp.float32),
        scratch_types=[
            pltpu.VMEM((NJ // 8, TPW, 8, 128), jnp.int32),
            pltpu.VMEM((128, D), jnp.float32),
            pltpu.VMEM((128, D), jnp.float32),
            pltpu.VMEM((2, 8, 129), jnp.float32),
            pltpu.VMEM((2, 8, 129), jnp.float32),
            pltpu.SemaphoreType.DMA,
            pltpu.SemaphoreType.DMA,
            pltpu.SemaphoreType.DMA,
            pltpu.SemaphoreType.DMA,
        ],
        compiler_params=pltpu.CompilerParams(
            use_tc_tiling_on_sc=False, needs_layout_passes=False),
    )
    def k(cb_hbm, idx_hbm, out_hbm, idx_v, rows0, rows1, tr0, tr1,
          sg0, sg1, so0, so1):
        rows = (rows0, rows1)
        trs = (tr0, tr1)
        sgs = (sg0, sg1)
        sos = (so0, so1)
        wid = lax.axis_index("s") * NC + lax.axis_index("c")
        t0 = wid * TPW

        def idx_slice(step):
            jl = step % NJ
            tl = step // NJ
            return idx_v.at[jl // 8, tl, jl % 8]

        pltpu.sync_copy(idx_hbm.at[:, pl.ds(t0, TPW)], idx_v)
        pltpu.async_copy(cb_hbm.at[idx_slice(0)], rows0, sg0)
        pltpu.async_copy(cb_hbm.at[idx_slice(1)], rows1, sg1)

        lanes = lax.iota(jnp.int32, 16)

        def body(outer, carry):
            for b in range(2):
                step = outer * 2 + b
                jl = step % NJ
                tg = t0 + step // NJ
                pltpu.make_async_copy(
                    cb_hbm.at[pl.ds(0, 128)], rows[b], sgs[b]).wait()
                @pl.when(step >= 2)
                def _():
                    pltpu.make_async_copy(
                        trs[b].at[:, :, pl.ds(0, 128)],
                        out_hbm.at[0, :, 0], sos[b]).wait()
                rb, tb = rows[b], trs[b]

                @plsc.parallel_loop(0, 128, 1, unroll=16)
                def _(i):
                    v = rb[i]
                    plsc.store_scatter(
                        tb, [lanes // 8, lanes % 8, jnp.full((16,), i, jnp.int32)], v)
                @pl.when(step + 2 < STEPS)
                def _():
                    pltpu.async_copy(
                        cb_hbm.at[idx_slice(step + 2)], rows[b], sgs[b])
                pltpu.async_copy(trs[b].at[:, :, pl.ds(0, 128)],
                                 out_hbm.at[jl, :, tg], sos[b])
            return carry

        lax.fori_loop(0, STEPS // 2, body, 0)
        pltpu.make_async_copy(
            tr0.at[:, :, pl.ds(0, 128)], out_hbm.at[0, :, 0], so0).wait()
        pltpu.make_async_copy(
            tr1.at[:, :, pl.ds(0, 128)], out_hbm.at[0, :, 0], so1).wait()

    return k


def kernel(data, codebook, indices):
    del data
    idx4 = (indices.astype(jnp.int32).T
            .reshape(NJ // 8, 8, 128, 128).transpose(0, 2, 1, 3))
    a = _make_gather()(codebook, idx4)
    return a.transpose(2, 4, 0, 1, 3).reshape(16384, NJ, D)
